# baseline (speedup 1.0000x reference)
import functools

import jax
import jax.numpy as jnp
from jax import lax
from jax.experimental import pallas as pl
from jax.experimental.pallas import tpu as pltpu
from jax.experimental.pallas import tpu_sc as plsc

N_CHUNK_ = 8
MAX_MASK_LEN_ = 256
SCALE_LOW_ = 0.5
SCALE_HIGH_ = 1.5

_NUM_CORES = 2
_NUM_SUBCORES = 16
_NW = _NUM_CORES * _NUM_SUBCORES
_LANES = 16


@functools.lru_cache(maxsize=None)
def _consts(B: int, L: int):
    import contextlib
    try:
        ctx = jax.default_device(jax.devices("cpu")[0])
    except RuntimeError:
        ctx = contextlib.nullcontext()
    with ctx:
        key = jax.random.key(42)
        kl, ks, kr = jax.random.split(key, 3)
        lengths = jax.random.randint(kl, (B, N_CHUNK_), 1, MAX_MASK_LEN_ + 1)
        starts = jax.random.randint(ks, (B, N_CHUNK_), 0, L)
        scale = (jax.random.uniform(kr, (B, L), dtype=jnp.float32)
                 * (SCALE_HIGH_ - SCALE_LOW_) + SCALE_LOW_)
        se = jnp.concatenate([starts, starts + lengths], axis=1)
        se = jnp.asarray(se, jnp.int32)
    return jax.device_get(se), jax.device_get(scale)


@functools.lru_cache(maxsize=None)
def _make_sc_call(B: int, L: int):
    chunk = B * L // _NW
    assert (B * L) % _NW == 0 and L % chunk == 0 and chunk % _LANES == 0
    nvec = chunk // _LANES

    mesh = plsc.VectorSubcoreMesh(
        core_axis_name="c", subcore_axis_name="s",
        num_cores=_NUM_CORES, num_subcores=_NUM_SUBCORES)

    @functools.partial(
        pl.kernel,
        out_type=jax.ShapeDtypeStruct((B, L), jnp.float32),
        mesh=mesh,
        scratch_types=[
            pltpu.VMEM((chunk,), jnp.float32),
            pltpu.VMEM((chunk,), jnp.float32),
            pltpu.VMEM((chunk,), jnp.int32),
            pltpu.VMEM((2 * N_CHUNK_,), jnp.int32),
        ],
    )
    def sc_call(t_h, s_h, v_h, se_h, out_h, t_v, s_v, v_v, se_v):
        wid = lax.axis_index("s") * _NUM_CORES + lax.axis_index("c")
        base = wid * chunk
        row = base // L
        col0 = base - row * L
        cols = pl.ds(col0, chunk)
        pltpu.sync_copy(t_h.at[row, cols], t_v)
        pltpu.sync_copy(s_h.at[row, cols], s_v)
        pltpu.sync_copy(v_h.at[row, cols], v_v)
        pltpu.sync_copy(se_h.at[row], se_v)
        lane = lax.iota(jnp.int32, _LANES)
        sev = se_v[pl.ds(0, 2 * N_CHUNK_)]
        for j in range(nvec):
            sl = pl.ds(j * _LANES, _LANES)
            pos = lane + (col0 + j * _LANES)
            hit = None
            for k in range(N_CHUNK_):
                m = (pos >= sev[k]) & (pos < sev[N_CHUNK_ + k])
                hit = m if hit is None else (hit | m)
            tv = t_v[sl]
            sel = hit & (v_v[sl] != 0)
            t_v[sl] = jnp.where(sel, tv * s_v[sl], tv)
        pltpu.sync_copy(t_v, out_h.at[row, cols])

    return sc_call


@functools.lru_cache(maxsize=None)
def _make_copy_call(shape, dtype_name):
    dtype = jnp.dtype(dtype_name)
    sds = jax.ShapeDtypeStruct(shape, dtype)

    def body(p_h, y_h, x_h, po_h, yo_h, xo_h, s0, s1, s2):
        c0 = pltpu.make_async_copy(p_h, po_h, s0)
        c1 = pltpu.make_async_copy(y_h, yo_h, s1)
        c2 = pltpu.make_async_copy(x_h, xo_h, s2)
        c0.start()
        c1.start()
        c2.start()
        c0.wait()
        c1.wait()
        c2.wait()

    return pl.pallas_call(
        body,
        out_shape=(sds, sds, sds),
        in_specs=[pl.BlockSpec(memory_space=pl.ANY)] * 3,
        out_specs=(pl.BlockSpec(memory_space=pl.ANY),) * 3,
        scratch_shapes=[pltpu.SemaphoreType.DMA] * 3,
    )


def kernel(p, y, x, t, valid_mask):
    B, L = t.shape
    se, scale = _consts(B, L)
    sc_call = _make_sc_call(B, L)
    t_new = sc_call(
        t,
        jnp.asarray(scale),
        valid_mask.astype(jnp.int32),
        jnp.asarray(se),
    )
    copy_call = _make_copy_call(p.shape, p.dtype.name)
    p2, y2, x2 = copy_call(p, y, x)
    return (p2, y2, x2, t_new, valid_mask)


# speedup vs baseline: 36.0492x; 36.0492x over previous
import functools

import jax
import jax.numpy as jnp
from jax import lax
from jax.experimental import pallas as pl
from jax.experimental.pallas import tpu as pltpu
from jax.experimental.pallas import tpu_sc as plsc

N_CHUNK_ = 8
MAX_MASK_LEN_ = 256
SCALE_LOW_ = 0.5
SCALE_HIGH_ = 1.5

_NUM_CORES = 2
_NUM_SUBCORES = 16
_NW = _NUM_CORES * _NUM_SUBCORES
_LANES = 16


@functools.lru_cache(maxsize=None)
def _consts(B: int, L: int):
    import contextlib
    try:
        ctx = jax.default_device(jax.devices("cpu")[0])
    except RuntimeError:
        ctx = contextlib.nullcontext()
    with ctx:
        key = jax.random.key(42)
        kl, ks, kr = jax.random.split(key, 3)
        lengths = jax.random.randint(kl, (B, N_CHUNK_), 1, MAX_MASK_LEN_ + 1)
        starts = jax.random.randint(ks, (B, N_CHUNK_), 0, L)
        scale = (jax.random.uniform(kr, (B, L), dtype=jnp.float32)
                 * (SCALE_HIGH_ - SCALE_LOW_) + SCALE_LOW_)
        se = jnp.concatenate([starts, starts + lengths], axis=1)
        se = jnp.asarray(se, jnp.int32)
    return jax.device_get(se), jax.device_get(scale)


@functools.lru_cache(maxsize=None)
def _make_sc_call(B: int, L: int):
    chunk = B * L // _NW
    assert (B * L) % _NW == 0 and L % chunk == 0 and chunk % _LANES == 0
    nvec = chunk // _LANES

    mesh = plsc.VectorSubcoreMesh(
        core_axis_name="c", subcore_axis_name="s",
        num_cores=_NUM_CORES, num_subcores=_NUM_SUBCORES)

    @functools.partial(
        pl.kernel,
        out_type=jax.ShapeDtypeStruct((B, L), jnp.float32),
        mesh=mesh,
        scratch_types=[
            pltpu.VMEM((chunk,), jnp.float32),
            pltpu.VMEM((chunk,), jnp.float32),
            pltpu.VMEM((chunk,), jnp.int32),
            pltpu.VMEM((2 * N_CHUNK_,), jnp.int32),
        ],
    )
    def sc_call(t_h, s_h, v_h, se_h, out_h, t_v, s_v, v_v, se_v):
        wid = lax.axis_index("s") * _NUM_CORES + lax.axis_index("c")
        base = wid * chunk
        row = base // L
        col0 = base - row * L
        cols = pl.ds(col0, chunk)
        pltpu.sync_copy(t_h.at[row, cols], t_v)
        pltpu.sync_copy(s_h.at[row, cols], s_v)
        pltpu.sync_copy(v_h.at[row, cols], v_v)
        pltpu.sync_copy(se_h.at[row], se_v)
        lane = lax.iota(jnp.int32, _LANES)
        sev = se_v[pl.ds(0, 2 * N_CHUNK_)]
        for j in range(nvec):
            sl = pl.ds(j * _LANES, _LANES)
            pos = lane + (col0 + j * _LANES)
            hit = None
            for k in range(N_CHUNK_):
                m = (pos >= sev[k]) & (pos < sev[N_CHUNK_ + k])
                hit = m if hit is None else (hit | m)
            tv = t_v[sl]
            sel = hit & (v_v[sl] != 0)
            t_v[sl] = jnp.where(sel, tv * s_v[sl], tv)
        pltpu.sync_copy(t_v, out_h.at[row, cols])

    return sc_call


@functools.lru_cache(maxsize=None)
def _make_copy_call(shape, dtype_name: str):
    dtype = jnp.dtype(dtype_name)
    B, L, D = shape
    blk_l = 256
    assert L % blk_l == 0
    grid = (B, L // blk_l)
    sds = jax.ShapeDtypeStruct(shape, dtype)
    spec = pl.BlockSpec((1, blk_l, D), lambda i, j: (i, j, 0))

    def body(p_v, y_v, x_v, po_v, yo_v, xo_v):
        po_v[...] = p_v[...]
        yo_v[...] = y_v[...]
        xo_v[...] = x_v[...]

    return pl.pallas_call(
        body,
        grid=grid,
        out_shape=(sds, sds, sds),
        in_specs=[spec] * 3,
        out_specs=(spec,) * 3,
    )


def kernel(p, y, x, t, valid_mask):
    B, L = t.shape
    se, scale = _consts(B, L)
    sc_call = _make_sc_call(B, L)
    t_new = sc_call(
        t,
        jnp.asarray(scale),
        valid_mask.astype(jnp.int32),
        jnp.asarray(se),
    )
    copy_call = _make_copy_call(p.shape, p.dtype.name)
    p2, y2, x2 = copy_call(p, y, x)
    return (p2, y2, x2, t_new, valid_mask)


# speedup vs baseline: 41.1448x; 1.1414x over previous
import functools

import jax
import jax.numpy as jnp
from jax import lax
from jax.experimental import pallas as pl
from jax.experimental.pallas import tpu as pltpu
from jax.experimental.pallas import tpu_sc as plsc

N_CHUNK_ = 8
MAX_MASK_LEN_ = 256
SCALE_LOW_ = 0.5
SCALE_HIGH_ = 1.5

_NUM_CORES = 2
_NUM_SUBCORES = 16
_NW = _NUM_CORES * _NUM_SUBCORES
_LANES = 16


@functools.lru_cache(maxsize=None)
def _consts(B: int, L: int):
    import contextlib
    try:
        ctx = jax.default_device(jax.devices("cpu")[0])
    except RuntimeError:
        ctx = contextlib.nullcontext()
    with ctx, jax.ensure_compile_time_eval():
        key = jax.random.key(42)
        kl, ks, kr = jax.random.split(key, 3)
        lengths = jax.random.randint(kl, (B, N_CHUNK_), 1, MAX_MASK_LEN_ + 1)
        starts = jax.random.randint(ks, (B, N_CHUNK_), 0, L)
        scale = (jax.random.uniform(kr, (B, L), dtype=jnp.float32)
                 * (SCALE_HIGH_ - SCALE_LOW_) + SCALE_LOW_)
        se = jnp.concatenate([starts, starts + lengths], axis=1)
        se = jnp.asarray(se, jnp.int32)
    return jax.device_get(se), jax.device_get(scale)


@functools.lru_cache(maxsize=None)
def _make_sc_call(B: int, L: int):
    chunk = B * L // _NW
    assert (B * L) % _NW == 0 and L % chunk == 0 and chunk % _LANES == 0
    nvec = chunk // _LANES

    mesh = plsc.VectorSubcoreMesh(
        core_axis_name="c", subcore_axis_name="s",
        num_cores=_NUM_CORES, num_subcores=_NUM_SUBCORES)

    @functools.partial(
        pl.kernel,
        out_type=jax.ShapeDtypeStruct((B, L), jnp.float32),
        mesh=mesh,
        scratch_types=[
            pltpu.VMEM((chunk,), jnp.float32),
            pltpu.VMEM((chunk,), jnp.float32),
            pltpu.VMEM((chunk,), jnp.int32),
            pltpu.VMEM((2 * N_CHUNK_,), jnp.int32),
        ],
    )
    def sc_call(t_h, s_h, v_h, se_h, out_h, t_v, s_v, v_v, se_v):
        wid = lax.axis_index("s") * _NUM_CORES + lax.axis_index("c")
        base = wid * chunk
        row = base // L
        col0 = base - row * L
        cols = pl.ds(col0, chunk)
        pltpu.sync_copy(t_h.at[row, cols], t_v)
        pltpu.sync_copy(s_h.at[row, cols], s_v)
        pltpu.sync_copy(v_h.at[row, cols], v_v)
        pltpu.sync_copy(se_h.at[row], se_v)
        lane = lax.iota(jnp.int32, _LANES)
        sev = se_v[pl.ds(0, 2 * N_CHUNK_)]
        for j in range(nvec):
            sl = pl.ds(j * _LANES, _LANES)
            pos = lane + (col0 + j * _LANES)
            hit = None
            for k in range(N_CHUNK_):
                m = (pos >= sev[k]) & (pos < sev[N_CHUNK_ + k])
                hit = m if hit is None else (hit | m)
            tv = t_v[sl]
            sel = hit & (v_v[sl] != 0)
            t_v[sl] = jnp.where(sel, tv * s_v[sl], tv)
        pltpu.sync_copy(t_v, out_h.at[row, cols])

    return sc_call


@functools.lru_cache(maxsize=None)
def _make_copy_call(shape, dtype_name: str):
    dtype = jnp.dtype(dtype_name)
    B, L, D = shape
    blk_l = 256
    assert L % blk_l == 0
    grid = (B, L // blk_l)
    sds = jax.ShapeDtypeStruct(shape, dtype)
    spec = pl.BlockSpec((1, blk_l, D), lambda i, j: (i, j, 0))

    def body(p_v, y_v, x_v, po_v, yo_v, xo_v):
        po_v[...] = p_v[...]
        yo_v[...] = y_v[...]
        xo_v[...] = x_v[...]

    return pl.pallas_call(
        body,
        grid=grid,
        out_shape=(sds, sds, sds),
        in_specs=[spec] * 3,
        out_specs=(spec,) * 3,
    )


def kernel(p, y, x, t, valid_mask):
    B, L = t.shape
    se, scale = _consts(B, L)
    sc_call = _make_sc_call(B, L)
    t_new = sc_call(
        t,
        jnp.asarray(scale),
        valid_mask.astype(jnp.int32),
        jnp.asarray(se),
    )
    copy_call = _make_copy_call(p.shape, p.dtype.name)
    p2, y2, x2 = copy_call(p, y, x)
    return (p2, y2, x2, t_new, valid_mask)


# speedup vs baseline: 42.4098x; 1.0307x over previous
import functools

import jax
import jax.numpy as jnp
from jax import lax
from jax.experimental import pallas as pl
from jax.experimental.pallas import tpu as pltpu
from jax.experimental.pallas import tpu_sc as plsc

N_CHUNK_ = 8
MAX_MASK_LEN_ = 256
SCALE_LOW_ = 0.5
SCALE_HIGH_ = 1.5

_NUM_CORES = 2
_NUM_SUBCORES = 16
_NW = _NUM_CORES * _NUM_SUBCORES
_LANES = 16


@functools.lru_cache(maxsize=None)
def _consts(B: int, L: int):
    import contextlib
    try:
        ctx = jax.default_device(jax.devices("cpu")[0])
    except RuntimeError:
        ctx = contextlib.nullcontext()
    with ctx, jax.ensure_compile_time_eval():
        key = jax.random.key(42)
        kl, ks, kr = jax.random.split(key, 3)
        lengths = jax.random.randint(kl, (B, N_CHUNK_), 1, MAX_MASK_LEN_ + 1)
        starts = jax.random.randint(ks, (B, N_CHUNK_), 0, L)
        scale = (jax.random.uniform(kr, (B, L), dtype=jnp.float32)
                 * (SCALE_HIGH_ - SCALE_LOW_) + SCALE_LOW_)
        se = jnp.concatenate([starts, starts + lengths], axis=1)
        se = jnp.asarray(se, jnp.int32)
    return jax.device_get(se), jax.device_get(scale)


@functools.lru_cache(maxsize=None)
def _make_sc_call(B: int, L: int):
    chunk = B * L // _NW
    assert (B * L) % _NW == 0 and L % chunk == 0 and chunk % _LANES == 0
    nvec = chunk // _LANES

    mesh = plsc.VectorSubcoreMesh(
        core_axis_name="c", subcore_axis_name="s",
        num_cores=_NUM_CORES, num_subcores=_NUM_SUBCORES)

    @functools.partial(
        pl.kernel,
        out_type=jax.ShapeDtypeStruct((B, L), jnp.float32),
        mesh=mesh,
        scratch_types=[
            pltpu.VMEM((chunk,), jnp.float32),
            pltpu.VMEM((chunk,), jnp.float32),
            pltpu.VMEM((chunk,), jnp.int32),
            pltpu.VMEM((2 * N_CHUNK_,), jnp.int32),
        ],
    )
    def sc_call(t_h, s_h, v_h, se_h, out_h, t_v, s_v, v_v, se_v):
        wid = lax.axis_index("s") * _NUM_CORES + lax.axis_index("c")
        base = wid * chunk
        row = base // L
        col0 = base - row * L
        cols = pl.ds(col0, chunk)
        pltpu.sync_copy(t_h.at[row, cols], t_v)
        pltpu.sync_copy(s_h.at[row, cols], s_v)
        pltpu.sync_copy(v_h.at[row, cols], v_v)
        pltpu.sync_copy(se_h.at[row], se_v)
        lane = lax.iota(jnp.int32, _LANES)
        sev = se_v[pl.ds(0, 2 * N_CHUNK_)]
        for j in range(nvec):
            sl = pl.ds(j * _LANES, _LANES)
            pos = lane + (col0 + j * _LANES)
            hit = None
            for k in range(N_CHUNK_):
                m = (pos >= sev[k]) & (pos < sev[N_CHUNK_ + k])
                hit = m if hit is None else (hit | m)
            tv = t_v[sl]
            sel = hit & (v_v[sl] != 0)
            t_v[sl] = jnp.where(sel, tv * s_v[sl], tv)
        pltpu.sync_copy(t_v, out_h.at[row, cols])

    return sc_call


@functools.lru_cache(maxsize=None)
def _make_copy_call(shape, dtype_name: str):
    dtype = jnp.dtype(dtype_name)
    B, L, D = shape
    blk_l = 512
    assert L % blk_l == 0
    grid = (B, L // blk_l)
    sds = jax.ShapeDtypeStruct(shape, dtype)
    spec = pl.BlockSpec((1, blk_l, D), lambda i, j: (i, j, 0))

    def body(p_v, y_v, x_v, po_v, yo_v, xo_v):
        po_v[...] = p_v[...]
        yo_v[...] = y_v[...]
        xo_v[...] = x_v[...]

    return pl.pallas_call(
        body,
        grid=grid,
        out_shape=(sds, sds, sds),
        in_specs=[spec] * 3,
        out_specs=(spec,) * 3,
    )


def kernel(p, y, x, t, valid_mask):
    B, L = t.shape
    se, scale = _consts(B, L)
    sc_call = _make_sc_call(B, L)
    t_new = sc_call(
        t,
        jnp.asarray(scale),
        valid_mask.astype(jnp.int32),
        jnp.asarray(se),
    )
    copy_call = _make_copy_call(p.shape, p.dtype.name)
    p2, y2, x2 = copy_call(p, y, x)
    return (p2, y2, x2, t_new, valid_mask)


# speedup vs baseline: 42.5906x; 1.0043x over previous
import functools

import jax
import jax.numpy as jnp
from jax import lax
from jax.experimental import pallas as pl
from jax.experimental.pallas import tpu as pltpu
from jax.experimental.pallas import tpu_sc as plsc

N_CHUNK_ = 8
MAX_MASK_LEN_ = 256
SCALE_LOW_ = 0.5
SCALE_HIGH_ = 1.5

_NUM_CORES = 2
_NUM_SUBCORES = 16
_NW = _NUM_CORES * _NUM_SUBCORES
_LANES = 16


@functools.lru_cache(maxsize=None)
def _consts(B: int, L: int):
    import contextlib
    try:
        ctx = jax.default_device(jax.devices("cpu")[0])
    except RuntimeError:
        ctx = contextlib.nullcontext()
    with ctx, jax.ensure_compile_time_eval():
        key = jax.random.key(42)
        kl, ks, kr = jax.random.split(key, 3)
        lengths = jax.random.randint(kl, (B, N_CHUNK_), 1, MAX_MASK_LEN_ + 1)
        starts = jax.random.randint(ks, (B, N_CHUNK_), 0, L)
        scale = (jax.random.uniform(kr, (B, L), dtype=jnp.float32)
                 * (SCALE_HIGH_ - SCALE_LOW_) + SCALE_LOW_)
        se = jnp.concatenate([starts, starts + lengths], axis=1)
        se = jnp.asarray(se, jnp.int32)
    return jax.device_get(se), jax.device_get(scale)


@functools.lru_cache(maxsize=None)
def _make_sc_call(B: int, L: int):
    chunk = B * L // _NW
    assert (B * L) % _NW == 0 and L % chunk == 0 and chunk % _LANES == 0
    nvec = chunk // _LANES

    mesh = plsc.VectorSubcoreMesh(
        core_axis_name="c", subcore_axis_name="s",
        num_cores=_NUM_CORES, num_subcores=_NUM_SUBCORES)

    @functools.partial(
        pl.kernel,
        out_type=jax.ShapeDtypeStruct((B, L), jnp.float32),
        mesh=mesh,
        scratch_types=[
            pltpu.VMEM((chunk,), jnp.float32),
            pltpu.VMEM((chunk,), jnp.float32),
            pltpu.VMEM((chunk,), jnp.int32),
            pltpu.VMEM((2 * N_CHUNK_,), jnp.int32),
        ],
    )
    def sc_call(t_h, s_h, v_h, se_h, out_h, t_v, s_v, v_v, se_v):
        wid = lax.axis_index("s") * _NUM_CORES + lax.axis_index("c")
        base = wid * chunk
        row = base // L
        col0 = base - row * L
        cols = pl.ds(col0, chunk)
        pltpu.sync_copy(t_h.at[row, cols], t_v)
        pltpu.sync_copy(s_h.at[row, cols], s_v)
        pltpu.sync_copy(v_h.at[row, cols], v_v)
        pltpu.sync_copy(se_h.at[row], se_v)
        lane = lax.iota(jnp.int32, _LANES)
        sev = se_v[pl.ds(0, 2 * N_CHUNK_)]
        for j in range(nvec):
            sl = pl.ds(j * _LANES, _LANES)
            pos = lane + (col0 + j * _LANES)
            hit = None
            for k in range(N_CHUNK_):
                m = (pos >= sev[k]) & (pos < sev[N_CHUNK_ + k])
                hit = m if hit is None else (hit | m)
            tv = t_v[sl]
            sel = hit & (v_v[sl] != 0)
            t_v[sl] = jnp.where(sel, tv * s_v[sl], tv)
        pltpu.sync_copy(t_v, out_h.at[row, cols])

    return sc_call


@functools.lru_cache(maxsize=None)
def _make_copy_call(shape, dtype_name: str):
    dtype = jnp.dtype(dtype_name)
    B, L, D = shape
    blk_l = 1024
    assert L % blk_l == 0
    grid = (B, L // blk_l)
    sds = jax.ShapeDtypeStruct(shape, dtype)
    spec = pl.BlockSpec((1, blk_l, D), lambda i, j: (i, j, 0))

    def body(p_v, y_v, x_v, po_v, yo_v, xo_v):
        po_v[...] = p_v[...]
        yo_v[...] = y_v[...]
        xo_v[...] = x_v[...]

    return pl.pallas_call(
        body,
        grid=grid,
        out_shape=(sds, sds, sds),
        in_specs=[spec] * 3,
        out_specs=(spec,) * 3,
    )


def kernel(p, y, x, t, valid_mask):
    B, L = t.shape
    se, scale = _consts(B, L)
    sc_call = _make_sc_call(B, L)
    t_new = sc_call(
        t,
        jnp.asarray(scale),
        valid_mask.astype(jnp.int32),
        jnp.asarray(se),
    )
    copy_call = _make_copy_call(p.shape, p.dtype.name)
    p2, y2, x2 = copy_call(p, y, x)
    return (p2, y2, x2, t_new, valid_mask)
